# baseline (speedup 1.0000x reference)
import functools
import jax
import jax.numpy as jnp
from jax import lax
from jax.experimental import pallas as pl
from jax.experimental.pallas import tpu as pltpu
from jax.experimental.pallas import tpu_sc as plsc

D = 64
D_PAD = 128
SCALE = 8.0

NC = 2
NS = 16
NW = NC * NS
BATCH = 4096
SEQ = 200
ROWS_W = BATCH // NW
SPLIT = (128, 72)
NBUF = 4


def _emb_body(x_hbm, w_hbm, out_hbm, idx_v, rows_v, gsem, wsem):
    wid = lax.axis_index("s") * NC + lax.axis_index("c")
    b0 = wid * ROWS_W
    pltpu.sync_copy(x_hbm.at[pl.ds(b0, ROWS_W)], idx_v)

    def fire_gathers(i, p):
        off = 0
        for n in SPLIT:
            pltpu.async_copy(
                w_hbm.at[idx_v.at[i, pl.ds(off, n)]],
                rows_v.at[p, pl.ds(off, n)],
                gsem.at[p])
            off += n

    def drain_gathers(p):
        pltpu.make_async_copy(w_hbm.at[pl.ds(0, SEQ)], rows_v.at[p],
                              gsem.at[p]).wait()

    def fire_wb(i, p):
        pltpu.async_copy(rows_v.at[p, :, pl.ds(0, D)],
                         out_hbm.at[b0 + i, :, pl.ds(0, D)], wsem.at[p])

    def wait_wb(i, p):
        pltpu.make_async_copy(out_hbm.at[b0 + i, :, pl.ds(0, D)],
                              rows_v.at[p, :, pl.ds(0, D)], wsem.at[p]).wait()

    def scale_buf(p):
        def row(i, _):
            for j in range(D // 16):
                sl = pl.ds(j * 16, 16)
                rows_v[p, i, sl] = rows_v[p, i, sl] * SCALE
            return 0
        lax.fori_loop(0, SEQ, row, 0)

    for r in range(NBUF - 1):
        fire_gathers(r, r)

    def step(t, _):
        for r in range(NBUF):
            g = t * NBUF + r
            drain_gathers(r)
            scale_buf(r)
            gn = g + NBUF - 1
            q = (r + NBUF - 1) % NBUF

            @pl.when(gn < ROWS_W)
            def _fire_ahead():
                @pl.when(g >= 1)
                def _wait_prev_wb():
                    wait_wb(g - 1, q)
                fire_gathers(gn, q)

            fire_wb(g, r)
        return 0

    lax.fori_loop(0, ROWS_W // NBUF, step, 0)

    for k in range(NBUF):
        gg = ROWS_W - NBUF + k
        wait_wb(gg, gg % NBUF)


@functools.partial(jax.jit, static_argnames=())
def kernel(x, W):
    w_pad = lax.optimization_barrier(jnp.pad(W, ((0, 0), (0, D_PAD - D))))
    mesh = plsc.VectorSubcoreMesh(core_axis_name="c", subcore_axis_name="s")
    out = pl.kernel(
        _emb_body,
        mesh=mesh,
        compiler_params=pltpu.CompilerParams(
            use_tc_tiling_on_sc=False, needs_layout_passes=False),
        out_type=jax.ShapeDtypeStruct((BATCH, SEQ, D_PAD), jnp.float32),
        scratch_types=[
            pltpu.VMEM((ROWS_W, SEQ), jnp.int32),
            pltpu.VMEM((NBUF, SEQ, D_PAD), jnp.float32),
            pltpu.SemaphoreType.DMA((NBUF,)),
            pltpu.SemaphoreType.DMA((NBUF,)),
        ],
    )(x.astype(jnp.int32), w_pad)
    return out[:, :, :D]


# speedup vs baseline: 1.0496x; 1.0496x over previous
import functools
import jax
import jax.numpy as jnp
from jax import lax
from jax.experimental import pallas as pl
from jax.experimental.pallas import tpu as pltpu
from jax.experimental.pallas import tpu_sc as plsc

D = 64
D_PAD = 128
SCALE = 8.0

NC = 2
NS = 16
NW = NC * NS
BATCH = 4096
SEQ = 200
SEQ_PAD = 256
ROWS_W = BATCH // NW
SPLIT = (128, 72)
NBUF = 4


def _emb_body(x_hbm, w_hbm, out_hbm, idx_v, rows_v, gsem, wsem):
    wid = lax.axis_index("s") * NC + lax.axis_index("c")
    b0 = wid * ROWS_W
    pltpu.sync_copy(x_hbm.at[pl.ds(b0, ROWS_W)], idx_v)

    def fire_gathers(i, p):
        off = 0
        for n in SPLIT:
            pltpu.async_copy(
                w_hbm.at[idx_v.at[i, pl.ds(off, n)]],
                rows_v.at[p, pl.ds(off, n)],
                gsem.at[p])
            off += n

    def drain_gathers(i, p):
        pltpu.make_async_copy(out_hbm.at[b0 + i, :, pl.ds(0, D)],
                              rows_v.at[p], gsem.at[p]).wait()

    def fire_wb(i, p):
        pltpu.async_copy(rows_v.at[p], out_hbm.at[b0 + i, :, pl.ds(0, D)],
                         wsem.at[p])

    def wait_wb(i, p):
        pltpu.make_async_copy(out_hbm.at[b0 + i, :, pl.ds(0, D)],
                              rows_v.at[p], wsem.at[p]).wait()

    def scale_buf(p):
        def rows2(i, _):
            for u in range(2):
                for j in range(D // 16):
                    sl = pl.ds(j * 16, 16)
                    rows_v[p, i * 2 + u, sl] = rows_v[p, i * 2 + u, sl] * SCALE
            return 0
        lax.fori_loop(0, SEQ // 2, rows2, 0)

    for r in range(NBUF - 1):
        fire_gathers(r, r)

    def step(t, _):
        for r in range(NBUF):
            g = t * NBUF + r
            drain_gathers(g, r)
            scale_buf(r)
            gn = g + NBUF - 1
            q = (r + NBUF - 1) % NBUF

            @pl.when(gn < ROWS_W)
            def _fire_ahead():
                @pl.when(g >= 1)
                def _wait_prev_wb():
                    wait_wb(g - 1, q)
                fire_gathers(gn, q)

            fire_wb(g, r)
        return 0

    lax.fori_loop(0, ROWS_W // NBUF, step, 0)

    for k in range(NBUF):
        gg = ROWS_W - NBUF + k
        wait_wb(gg, gg % NBUF)


@functools.partial(jax.jit, static_argnames=())
def kernel(x, W):
    x_pad = jnp.pad(x.astype(jnp.int32), ((0, 0), (0, SEQ_PAD - SEQ)))
    mesh = plsc.VectorSubcoreMesh(core_axis_name="c", subcore_axis_name="s")
    out = pl.kernel(
        _emb_body,
        mesh=mesh,
        compiler_params=pltpu.CompilerParams(use_tc_tiling_on_sc=False),
        out_type=jax.ShapeDtypeStruct((BATCH, SEQ, D_PAD), jnp.float32),
        scratch_types=[
            pltpu.VMEM((ROWS_W, SEQ_PAD), jnp.int32),
            pltpu.VMEM((NBUF, SEQ, D), jnp.float32),
            pltpu.SemaphoreType.DMA((NBUF,)),
            pltpu.SemaphoreType.DMA((NBUF,)),
        ],
    )(x_pad, W)
    return out[:, :, :D]
